# baseline (speedup 1.0000x reference)
import functools

import jax
import jax.numpy as jnp
from jax import lax
from jax.experimental import pallas as pl
from jax.experimental.pallas import tpu as pltpu
from jax.experimental.pallas import tpu_sc as plsc

NC = 2
NS = 16
L = 16
NW = NC * NS

BATCH = 16384
K = 16
KP = K // 2
B_PER_W = BATCH // NW
CHUNK = 128
N_CHUNK = B_PER_W // CHUNK
N_BLOCK = B_PER_W // L


def _mf_kernel(iu_hbm, ip_hbm, uemb_hbm, pemb_hbm, w_hbm, b_hbm, out_hbm,
               idxu_v, idxp_v, rows_u, rows_p, out_v, w_v, b_v, sem):
    wid = lax.axis_index("s") * NC + lax.axis_index("c")
    base_chunk = wid * N_CHUNK
    out_base = wid * B_PER_W

    pltpu.sync_copy(iu_hbm.at[pl.ds(base_chunk, N_CHUNK)], idxu_v)
    pltpu.sync_copy(ip_hbm.at[pl.ds(base_chunk, N_CHUNK)], idxp_v)
    pltpu.sync_copy(w_hbm, w_v)
    pltpu.sync_copy(b_hbm, b_v)

    copies = []
    for c in range(N_CHUNK):
        copies.append(pltpu.async_copy(
            uemb_hbm.at[idxu_v.at[c]], rows_u.at[pl.ds(c * CHUNK, CHUNK)],
            sem))
        copies.append(pltpu.async_copy(
            pemb_hbm.at[idxp_v.at[c]], rows_p.at[pl.ds(c * CHUNK, CHUNK)],
            sem))
    for cp in copies:
        cp.wait()

    iota = lax.iota(jnp.int32, L)
    col_ids = [jnp.full((L,), k, jnp.int32) for k in range(KP)]
    wk_vecs = [w_v[k, :] for k in range(K)]
    bias = b_v[...]

    def block(blk, _):
        row_idx = blk * L + iota
        acc = bias
        for k in range(KP):
            up = plsc.load_gather(rows_u, [row_idx, col_ids[k]])
            pp = plsc.load_gather(rows_p, [row_idx, col_ids[k]])
            ue, uo = plsc.unpack(plsc.bitcast(up, jnp.bfloat16),
                                 format=plsc.PackFormat.INTERLEAVED)
            pe, po = plsc.unpack(plsc.bitcast(pp, jnp.bfloat16),
                                 format=plsc.PackFormat.INTERLEAVED)
            acc = acc + (ue * pe) * wk_vecs[2 * k]
            acc = acc + (uo * po) * wk_vecs[2 * k + 1]
        out_v[pl.ds(blk * L, L)] = acc
        return 0

    lax.fori_loop(0, N_BLOCK, block, 0)

    pltpu.sync_copy(out_v, out_hbm.at[pl.ds(out_base, B_PER_W)])


@jax.jit
def _mf(iu, ip, upk, ppk, w_flat, b_vec):
    run = pl.kernel(
        _mf_kernel,
        out_type=jax.ShapeDtypeStruct((BATCH,), jnp.float32),
        mesh=plsc.VectorSubcoreMesh(core_axis_name="c", subcore_axis_name="s",
                                    num_cores=NC, num_subcores=NS),
        compiler_params=pltpu.CompilerParams(needs_layout_passes=False,
                                             use_tc_tiling_on_sc=False),
        scratch_types=[
            pltpu.VMEM((N_CHUNK, CHUNK), jnp.int32),
            pltpu.VMEM((N_CHUNK, CHUNK), jnp.int32),
            pltpu.VMEM((B_PER_W, KP), jnp.int32),
            pltpu.VMEM((B_PER_W, KP), jnp.int32),
            pltpu.VMEM((B_PER_W,), jnp.float32),
            pltpu.VMEM((K, L), jnp.float32),
            pltpu.VMEM((L,), jnp.float32),
            pltpu.SemaphoreType.DMA,
        ],
    )
    return run(iu, ip, upk, ppk, w_flat, b_vec)


def _pack_table(table):
    bf = table.astype(jnp.bfloat16).reshape(table.shape[0], KP, 2)
    packed = lax.bitcast_convert_type(bf, jnp.int32)
    return lax.optimization_barrier(packed)


def kernel(input_user, input_prob, user_emb, prob_emb, dense_w, dense_b):
    iu = input_user.reshape(NW * N_CHUNK, CHUNK)
    ip = input_prob.reshape(NW * N_CHUNK, CHUNK)
    w_bcast = jnp.broadcast_to(dense_w.reshape(K, 1), (K, L))
    b_vec = jnp.broadcast_to(dense_b, (L,))
    out = _mf(iu, ip, _pack_table(user_emb), _pack_table(prob_emb),
              w_bcast, b_vec)
    return out.reshape(BATCH, 1)
